# baseline (speedup 1.0000x reference)
import functools

import jax
import jax.numpy as jnp
from jax import lax
from jax.experimental import pallas as pl
from jax.experimental.pallas import tpu as pltpu
from jax.experimental.pallas import tpu_sc as plsc

N = 10000
C = 10000
E = 320000
D = 128

NC = 2
NS = 16
NW = NC * NS

K = 128
NJ = 81
NCH = NW * NJ
EPAD = NCH * K - E
NTRI = NJ // 3
ROWS_PER_TILE = 632
CPAD = NS * ROWS_PER_TILE


def _sc_body(x_hbm, inc_hbm, zeros_hbm, out_hbm,
             idx, rows, acc, *sems):
    semi = sems[0:3]
    semg = sems[3:6]
    cid = lax.axis_index("c")
    sid = lax.axis_index("s")
    wid = sid * NC + cid

    r0 = sid * ROWS_PER_TILE
    pltpu.sync_copy(zeros_hbm, acc.at[pl.ds(r0, ROWS_PER_TILE)])
    plsc.subcore_barrier()

    base = wid * NJ * K

    def idx_load(c, slot):
        pltpu.async_copy(inc_hbm.at[:, pl.ds(base + c * K, K)], idx.at[slot],
                         semi[slot])

    def idx_wait(slot):
        pltpu.make_async_copy(inc_hbm.at[:, pl.ds(0, K)], idx.at[slot],
                              semi[slot]).wait()

    def gather(islot, rslot):
        pltpu.async_copy(x_hbm.at[idx.at[islot, 0]], rows.at[rslot], semg[rslot])

    def gather_wait(rslot):
        pltpu.make_async_copy(x_hbm.at[pl.ds(0, K)], rows.at[rslot],
                              semg[rslot]).wait()

    def scatter(slot):
        pltpu.sync_copy(rows.at[slot], acc.at[idx.at[slot, 1]], add=True)

    idx_load(0, 0)
    idx_load(1, 1)
    idx_load(2, 2)
    idx_wait(0)
    gather(0, 0)
    idx_wait(1)
    gather(1, 1)

    def tri(j, carry):
        c = 3 * j
        for i in range(3):
            p = (i + 2) % 3
            gather_wait(i)

            @pl.when(c + 2 + i < NJ)
            def _():
                idx_wait(p)
                gather(p, p)

            scatter(i)

            @pl.when(j < NTRI - 1)
            def _():
                idx_load(c + 3 + i, i)
        return carry

    lax.fori_loop(0, NTRI, tri, jnp.int32(0))
    plsc.subcore_barrier()

    pltpu.sync_copy(acc.at[pl.ds(r0, ROWS_PER_TILE)],
                    out_hbm.at[cid, pl.ds(r0, ROWS_PER_TILE)])


_sc_segment_sum = functools.partial(
    pl.kernel,
    out_type=jax.ShapeDtypeStruct((NC, CPAD, D), jnp.float32),
    mesh=plsc.VectorSubcoreMesh(
        core_axis_name="c", subcore_axis_name="s", num_cores=NC, num_subcores=NS
    ),
    scratch_types=[
        pltpu.VMEM((3, 2, K), jnp.int32),
        pltpu.VMEM((3, K, D), jnp.float32),
        pltpu.VMEM_SHARED((CPAD, D), jnp.float32),
        pltpu.SemaphoreType.DMA,
        pltpu.SemaphoreType.DMA,
        pltpu.SemaphoreType.DMA,
        pltpu.SemaphoreType.DMA,
        pltpu.SemaphoreType.DMA,
        pltpu.SemaphoreType.DMA,
    ],
)(_sc_body)


BC = 2000


def _mlp_body(p_ref, w1_ref, b1_ref, w2_ref, b2_ref, o_ref):
    a = p_ref[0] + p_ref[1]
    h = jnp.dot(a, w1_ref[...], preferred_element_type=jnp.float32) + b1_ref[...]
    h = jnp.maximum(h, 0.0)
    o_ref[...] = (
        jnp.dot(h, w2_ref[...], preferred_element_type=jnp.float32) + b2_ref[...]
    )


_mlp = pl.pallas_call(
    _mlp_body,
    grid=(C // BC,),
    in_specs=[
        pl.BlockSpec((NC, BC, D), lambda i: (0, i, 0)),
        pl.BlockSpec((D, D), lambda i: (0, 0)),
        pl.BlockSpec((1, D), lambda i: (0, 0)),
        pl.BlockSpec((D, D), lambda i: (0, 0)),
        pl.BlockSpec((1, D), lambda i: (0, 0)),
    ],
    out_specs=pl.BlockSpec((BC, D), lambda i: (i, 0)),
    out_shape=jax.ShapeDtypeStruct((C, D), jnp.float32),
)


def kernel(x, incidence_index, W1, b1, W2, b2, eps):
    inc = incidence_index.astype(jnp.int32)
    pad_dst = C + (jnp.arange(EPAD, dtype=jnp.int32) % (CPAD - C))
    pad = jnp.stack([jnp.zeros((EPAD,), jnp.int32), pad_dst], axis=0)
    inc2 = jnp.concatenate([inc, pad], axis=1)
    zeros = jnp.zeros((ROWS_PER_TILE, D), dtype=jnp.float32)
    partials = _sc_segment_sum(x, inc2, zeros)
    w1s = W1 * (1.0 + eps)
    return _mlp(partials, w1s, b1.reshape(1, D), W2, b2.reshape(1, D))


# speedup vs baseline: 1.1046x; 1.1046x over previous
import functools

import jax
import jax.numpy as jnp
from jax import lax
from jax.experimental import pallas as pl
from jax.experimental.pallas import tpu as pltpu
from jax.experimental.pallas import tpu_sc as plsc

N = 10000
C = 10000
E = 320000
D = 128

NC = 2
NS = 16
NW = NC * NS

K = 128
NJ = 80
NCH = NW * NJ
EPAD = NCH * K - E
ROWS_PER_TILE = 632
CPAD = NS * ROWS_PER_TILE


def _sc_body(x_hbm, inc_hbm, zeros_hbm, out_hbm,
             srcA, dstA, srcB, dstB, rows, acc, semA, semB):
    cid = lax.axis_index("c")
    sid = lax.axis_index("s")
    wid = sid * NC + cid

    r0 = sid * ROWS_PER_TILE
    pltpu.sync_copy(zeros_hbm, acc.at[pl.ds(r0, ROWS_PER_TILE)])
    plsc.subcore_barrier()

    base = wid * NJ * K

    def pair(j, carry):
        bA = base + (2 * j) * K
        bB = bA + K
        pltpu.sync_copy(inc_hbm.at[0, pl.ds(bA, K)], srcA)
        pltpu.sync_copy(inc_hbm.at[1, pl.ds(bA, K)], dstA)
        pltpu.sync_copy(inc_hbm.at[0, pl.ds(bB, K)], srcB)
        pltpu.sync_copy(inc_hbm.at[1, pl.ds(bB, K)], dstB)
        gA = pltpu.async_copy(x_hbm.at[srcA], rows.at[0], semA)
        gB = pltpu.async_copy(x_hbm.at[srcB], rows.at[1], semB)
        gA.wait()
        pltpu.sync_copy(rows.at[0], acc.at[dstA], add=True)
        gB.wait()
        pltpu.sync_copy(rows.at[1], acc.at[dstB], add=True)
        return carry

    lax.fori_loop(0, NJ // 2, pair, jnp.int32(0))
    plsc.subcore_barrier()

    pltpu.sync_copy(acc.at[pl.ds(r0, ROWS_PER_TILE)],
                    out_hbm.at[cid, pl.ds(r0, ROWS_PER_TILE)])


_sc_segment_sum = functools.partial(
    pl.kernel,
    out_type=jax.ShapeDtypeStruct((NC, CPAD, D), jnp.float32),
    mesh=plsc.VectorSubcoreMesh(
        core_axis_name="c", subcore_axis_name="s", num_cores=NC, num_subcores=NS
    ),
    scratch_types=[
        pltpu.VMEM((K,), jnp.int32),
        pltpu.VMEM((K,), jnp.int32),
        pltpu.VMEM((K,), jnp.int32),
        pltpu.VMEM((K,), jnp.int32),
        pltpu.VMEM((2, K, D), jnp.float32),
        pltpu.VMEM_SHARED((CPAD, D), jnp.float32),
        pltpu.SemaphoreType.DMA,
        pltpu.SemaphoreType.DMA,
    ],
)(_sc_body)


BC = 2000


def _mlp_body(p_ref, w1_ref, b1_ref, w2_ref, b2_ref, o_ref):
    a = p_ref[0] + p_ref[1]
    h = jnp.dot(a, w1_ref[...], preferred_element_type=jnp.float32) + b1_ref[...]
    h = jnp.maximum(h, 0.0)
    o_ref[...] = (
        jnp.dot(h, w2_ref[...], preferred_element_type=jnp.float32) + b2_ref[...]
    )


_mlp = pl.pallas_call(
    _mlp_body,
    grid=(C // BC,),
    in_specs=[
        pl.BlockSpec((NC, BC, D), lambda i: (0, i, 0)),
        pl.BlockSpec((D, D), lambda i: (0, 0)),
        pl.BlockSpec((1, D), lambda i: (0, 0)),
        pl.BlockSpec((D, D), lambda i: (0, 0)),
        pl.BlockSpec((1, D), lambda i: (0, 0)),
    ],
    out_specs=pl.BlockSpec((BC, D), lambda i: (i, 0)),
    out_shape=jax.ShapeDtypeStruct((C, D), jnp.float32),
)


def kernel(x, incidence_index, W1, b1, W2, b2, eps):
    inc = incidence_index.astype(jnp.int32)
    pad_dst = C + (jnp.arange(EPAD, dtype=jnp.int32) % (CPAD - C))
    pad = jnp.stack([jnp.zeros((EPAD,), jnp.int32), pad_dst], axis=0)
    inc2 = jnp.concatenate([inc, pad], axis=1)
    zeros = jnp.zeros((ROWS_PER_TILE, D), dtype=jnp.float32)
    partials = _sc_segment_sum(x, inc2, zeros)
    w1s = W1 * (1.0 + eps)
    return _mlp(partials, w1s, b1.reshape(1, D), W2, b2.reshape(1, D))


# speedup vs baseline: 1.2521x; 1.1335x over previous
import functools

import jax
import jax.numpy as jnp
from jax import lax
from jax.experimental import pallas as pl
from jax.experimental.pallas import tpu as pltpu
from jax.experimental.pallas import tpu_sc as plsc

N = 10000
C = 10000
E = 320000
D = 128

NC = 2
NS = 16
NW = NC * NS

K = 128
NJ = 80
NCH = NW * NJ
EPAD = NCH * K - E
ROWS_PER_TILE = 632
CPAD = NS * ROWS_PER_TILE


def _sc_body(x_hbm, inc_hbm, zeros_hbm, out_hbm,
             srcA, dstA, srcB, dstB, rows, acc, semA, semB):
    cid = lax.axis_index("c")
    sid = lax.axis_index("s")
    wid = sid * NC + cid

    r0 = sid * ROWS_PER_TILE
    pltpu.sync_copy(zeros_hbm, acc.at[pl.ds(r0, ROWS_PER_TILE)])
    plsc.subcore_barrier()

    def pair(j, carry):
        bA = ((2 * j) * NW + wid) * K
        bB = ((2 * j + 1) * NW + wid) * K
        pltpu.sync_copy(inc_hbm.at[0, pl.ds(bA, K)], srcA)
        pltpu.sync_copy(inc_hbm.at[1, pl.ds(bA, K)], dstA)
        pltpu.sync_copy(inc_hbm.at[0, pl.ds(bB, K)], srcB)
        pltpu.sync_copy(inc_hbm.at[1, pl.ds(bB, K)], dstB)
        gA = pltpu.async_copy(x_hbm.at[srcA], rows.at[0], semA)
        gB = pltpu.async_copy(x_hbm.at[srcB], rows.at[1], semB)
        gA.wait()
        pltpu.sync_copy(rows.at[0], acc.at[dstA], add=True)
        gB.wait()
        pltpu.sync_copy(rows.at[1], acc.at[dstB], add=True)
        return carry

    lax.fori_loop(0, NJ // 2, pair, jnp.int32(0))
    plsc.subcore_barrier()

    pltpu.sync_copy(acc.at[pl.ds(r0, ROWS_PER_TILE)],
                    out_hbm.at[cid, pl.ds(r0, ROWS_PER_TILE)])


_sc_segment_sum = functools.partial(
    pl.kernel,
    out_type=jax.ShapeDtypeStruct((NC, CPAD, D), jnp.float32),
    mesh=plsc.VectorSubcoreMesh(
        core_axis_name="c", subcore_axis_name="s", num_cores=NC, num_subcores=NS
    ),
    scratch_types=[
        pltpu.VMEM((K,), jnp.int32),
        pltpu.VMEM((K,), jnp.int32),
        pltpu.VMEM((K,), jnp.int32),
        pltpu.VMEM((K,), jnp.int32),
        pltpu.VMEM((2, K, D), jnp.float32),
        pltpu.VMEM_SHARED((CPAD, D), jnp.float32),
        pltpu.SemaphoreType.DMA,
        pltpu.SemaphoreType.DMA,
    ],
)(_sc_body)


BC = 2000


def _mlp_body(p_ref, w1_ref, b1_ref, w2_ref, b2_ref, o_ref):
    a = p_ref[0] + p_ref[1]
    h = jnp.dot(a, w1_ref[...], preferred_element_type=jnp.float32) + b1_ref[...]
    h = jnp.maximum(h, 0.0)
    o_ref[...] = (
        jnp.dot(h, w2_ref[...], preferred_element_type=jnp.float32) + b2_ref[...]
    )


_mlp = pl.pallas_call(
    _mlp_body,
    grid=(C // BC,),
    in_specs=[
        pl.BlockSpec((NC, BC, D), lambda i: (0, i, 0)),
        pl.BlockSpec((D, D), lambda i: (0, 0)),
        pl.BlockSpec((1, D), lambda i: (0, 0)),
        pl.BlockSpec((D, D), lambda i: (0, 0)),
        pl.BlockSpec((1, D), lambda i: (0, 0)),
    ],
    out_specs=pl.BlockSpec((BC, D), lambda i: (i, 0)),
    out_shape=jax.ShapeDtypeStruct((C, D), jnp.float32),
)


def kernel(x, incidence_index, W1, b1, W2, b2, eps):
    inc = incidence_index.astype(jnp.int32)
    pad_dst = C + (jnp.arange(EPAD, dtype=jnp.int32) % (CPAD - C))
    pad = jnp.stack([jnp.zeros((EPAD,), jnp.int32), pad_dst], axis=0)
    inc2 = jnp.concatenate([inc, pad], axis=1)
    zeros = jnp.zeros((ROWS_PER_TILE, D), dtype=jnp.float32)
    partials = _sc_segment_sum(x, inc2, zeros)
    w1s = W1 * (1.0 + eps)
    return _mlp(partials, w1s, b1.reshape(1, D), W2, b2.reshape(1, D))
